# speedup vs baseline: 10.9044x; 10.9044x over previous
import functools

import jax
import jax.numpy as jnp
from jax import lax
from jax.experimental import pallas as pl
from jax.experimental.pallas import tpu as pltpu
from jax.experimental.pallas import tpu_sc as plsc

N_NODES = 10000
N_EDGES = 320000
D = 128
NEG = 0.01

NPAD = 10240
NC, NS = 2, 16
NW = NC * NS
EPW = N_EDGES // NW
C = 80
ROWS_PER_TILE = NPAD // NS
DEG_W = 16

_MESH = plsc.VectorSubcoreMesh(core_axis_name="c", subcore_axis_name="s")


def _zero_vmem_2d(ref, nrows, ncols):
    z16 = jnp.zeros((16,), jnp.float32)
    per_row = ncols // 16

    def body(i, carry):
        r = i // per_row
        c = (i % per_row) * 16
        ref[r, pl.ds(c, 16)] = z16
        return carry

    lax.fori_loop(0, nrows * per_row, body, 0)


@functools.partial(
    pl.kernel,
    out_type=jax.ShapeDtypeStruct((NC, NPAD, DEG_W), jnp.float32),
    mesh=_MESH,
    scratch_types=[
        pltpu.VMEM((C,), jnp.int32),
        pltpu.VMEM((C, DEG_W), jnp.float32),
        pltpu.VMEM_SHARED((NPAD, DEG_W), jnp.float32),
    ],
)
def _sc_deg(dst_hbm, out_hbm, didx, ones, acc):
    cid = lax.axis_index("c")
    sid = lax.axis_index("s")
    wid = sid * NC + cid

    _zero_vmem_2d(ones, C, DEG_W)

    def zero_acc(j, carry):
        pltpu.sync_copy(ones, acc.at[pl.ds(sid * ROWS_PER_TILE + j * C, C)])
        return carry

    lax.fori_loop(0, ROWS_PER_TILE // C, zero_acc, 0)

    one16 = jnp.full((16,), 1.0, jnp.float32)

    def fill_ones(r, carry):
        ones[r, pl.ds(0, 16)] = one16
        return carry

    lax.fori_loop(0, C, fill_ones, 0)
    plsc.subcore_barrier()

    base0 = wid * EPW

    def step(k, carry):
        pltpu.sync_copy(dst_hbm.at[pl.ds(base0 + k * C, C)], didx)
        pltpu.sync_copy(ones, acc.at[didx], add=True)
        return carry

    lax.fori_loop(0, EPW // C, step, 0)
    plsc.subcore_barrier()

    pltpu.sync_copy(
        acc.at[pl.ds(sid * ROWS_PER_TILE, ROWS_PER_TILE)],
        out_hbm.at[cid, pl.ds(sid * ROWS_PER_TILE, ROWS_PER_TILE)],
    )


@functools.partial(
    pl.kernel,
    out_type=jax.ShapeDtypeStruct((NC, NPAD, D), jnp.float32),
    mesh=_MESH,
    scratch_types=[
        pltpu.VMEM((C,), jnp.int32),
        pltpu.VMEM((C,), jnp.int32),
        pltpu.VMEM((C, D), jnp.float32),
        pltpu.VMEM_SHARED((NPAD, D), jnp.float32),
        pltpu.SemaphoreType.DMA,
    ],
)
def _sc_scatter(g_hbm, src_hbm, dst_hbm, out_hbm, sidx, didx, rows, acc, sem):
    cid = lax.axis_index("c")
    sid = lax.axis_index("s")
    wid = sid * NC + cid

    _zero_vmem_2d(rows, C, D)

    def zero_acc(j, carry):
        pltpu.sync_copy(rows, acc.at[pl.ds(sid * ROWS_PER_TILE + j * C, C)])
        return carry

    lax.fori_loop(0, ROWS_PER_TILE // C, zero_acc, 0)
    plsc.subcore_barrier()

    base0 = wid * EPW

    def step(k, carry):
        base = base0 + k * C
        pltpu.sync_copy(src_hbm.at[pl.ds(base, C)], sidx)
        pltpu.sync_copy(dst_hbm.at[pl.ds(base, C)], didx)
        pltpu.async_copy(g_hbm.at[sidx], rows, sem).wait()
        pltpu.sync_copy(rows, acc.at[didx], add=True)
        return carry

    lax.fori_loop(0, EPW // C, step, 0)
    plsc.subcore_barrier()

    pltpu.sync_copy(
        acc.at[pl.ds(sid * ROWS_PER_TILE, ROWS_PER_TILE)],
        out_hbm.at[cid, pl.ds(sid * ROWS_PER_TILE, ROWS_PER_TILE)],
    )


BLK = 512
GRID = NPAD // BLK


def _dinv_from(deg_ref):
    deg = 1.0 + deg_ref[0, :, 0:1] + deg_ref[1, :, 0:1]
    return lax.rsqrt(deg)


def _tc_pre_body(x_ref, w_ref, deg_ref, h_ref, g_ref):
    h = jnp.dot(x_ref[...], w_ref[...], preferred_element_type=jnp.float32)
    dinv = _dinv_from(deg_ref)
    h_ref[...] = h
    g_ref[...] = h * dinv


def _tc_mid_body(s_ref, h_ref, deg_ref, b_ref, w_ref, hn_ref, gn_ref):
    dinv = _dinv_from(deg_ref)
    h = h_ref[...]
    v = dinv * (s_ref[0] + s_ref[1]) + (dinv * dinv) * h + b_ref[...]
    a = jnp.where(v >= 0, v, NEG * v)
    hn = jnp.dot(a, w_ref[...], preferred_element_type=jnp.float32)
    hn_ref[...] = hn
    gn_ref[...] = hn * dinv


def _tc_post_body(s_ref, h_ref, deg_ref, b_ref, o_ref):
    dinv = _dinv_from(deg_ref)
    h = h_ref[...]
    v = dinv * (s_ref[0] + s_ref[1]) + (dinv * dinv) * h + b_ref[...]
    o_ref[...] = jnp.where(v >= 0, v, NEG * v)


_ROWBLK = pl.BlockSpec((BLK, D), lambda i: (i, 0))
_WSPEC = pl.BlockSpec((D, D), lambda i: (0, 0))
_DEGSPEC = pl.BlockSpec((NC, BLK, DEG_W), lambda i: (0, i, 0))
_SSPEC = pl.BlockSpec((NC, BLK, D), lambda i: (0, i, 0))
_BSPEC = pl.BlockSpec((1, D), lambda i: (0, 0))
_F32ROW = jax.ShapeDtypeStruct((NPAD, D), jnp.float32)

_tc_pre = pl.pallas_call(
    _tc_pre_body,
    grid=(GRID,),
    in_specs=[_ROWBLK, _WSPEC, _DEGSPEC],
    out_specs=[_ROWBLK, _ROWBLK],
    out_shape=[_F32ROW, _F32ROW],
)

_tc_mid = pl.pallas_call(
    _tc_mid_body,
    grid=(GRID,),
    in_specs=[_SSPEC, _ROWBLK, _DEGSPEC, _BSPEC, _WSPEC],
    out_specs=[_ROWBLK, _ROWBLK],
    out_shape=[_F32ROW, _F32ROW],
)

_tc_post = pl.pallas_call(
    _tc_post_body,
    grid=(GRID,),
    in_specs=[_SSPEC, _ROWBLK, _DEGSPEC, _BSPEC],
    out_specs=_ROWBLK,
    out_shape=_F32ROW,
)


def kernel(x, edge_index, W1, b1, W2, b2, W3, b3):
    src = edge_index[0]
    dst = edge_index[1]
    x_pad = jnp.zeros((NPAD, D), jnp.float32).at[:N_NODES].set(x)
    b1r = b1.reshape(1, D)
    b2r = b2.reshape(1, D)
    b3r = b3.reshape(1, D)

    degp = _sc_deg(dst)
    h1, g1 = _tc_pre(x_pad, W1, degp)
    s1 = _sc_scatter(g1, src, dst)
    h2, g2 = _tc_mid(s1, h1, degp, b1r, W2)
    s2 = _sc_scatter(g2, src, dst)
    h3, g3 = _tc_mid(s2, h2, degp, b2r, W3)
    s3 = _sc_scatter(g3, src, dst)
    out = _tc_post(s3, h3, degp, b3r)
    return out[:N_NODES]
